# baseline (speedup 1.0000x reference)
import functools

import jax
import jax.numpy as jnp
from jax import lax
from jax.experimental import pallas as pl
from jax.experimental.pallas import tpu as pltpu
from jax.experimental.pallas import tpu_sc as plsc

N_ATM = 10000
N_RES = 1000

NC = 2
NS = 16
NW = NC * NS
C = 128

_MESH = dict(core_axis_name="c", subcore_axis_name="s")


def _row_add(bufr, bufc, outb, nrows, D):
    def rbody(i, carry):
        for j in range(D // 16):
            sl = pl.ds(j * 16, 16)
            outb[i, sl] = bufr[i, sl] + bufc[i, sl]
        return carry
    lax.fori_loop(0, nrows, rbody, 0)


@functools.lru_cache(maxsize=None)
def _make_gather2(E, D):
    Ew = E // NW
    assert Ew * NW == E
    nfull, rem = divmod(Ew, C)
    nsteady = nfull if nfull % 2 == 0 else nfull - 1
    scratch = []
    for _ in range(2):
        scratch += [pltpu.VMEM((C,), jnp.int32), pltpu.VMEM((C,), jnp.int32),
                    pltpu.VMEM((C, D), jnp.float32), pltpu.VMEM((C, D), jnp.float32),
                    pltpu.VMEM((C, D), jnp.float32),
                    pltpu.SemaphoreType.DMA, pltpu.SemaphoreType.DMA,
                    pltpu.SemaphoreType.DMA]
    if rem:
        scratch += [pltpu.VMEM((rem,), jnp.int32), pltpu.VMEM((rem,), jnp.int32),
                    pltpu.VMEM((rem, D), jnp.float32), pltpu.VMEM((rem, D), jnp.float32),
                    pltpu.VMEM((rem, D), jnp.float32)]

    def body(tr_hbm, tc_hbm, row_hbm, col_hbm, out_hbm, *s):
        bufs = [s[0:8], s[8:16]]
        tail = s[16:] if rem else None
        wid = lax.axis_index("s") * NC + lax.axis_index("c")
        base = wid * Ew

        def issue_idx(g, p):
            ir, ic = bufs[p][0], bufs[p][1]
            off = base + g * C
            d1 = pltpu.async_copy(row_hbm.at[pl.ds(off, C)], ir, bufs[p][5])
            d2 = pltpu.async_copy(col_hbm.at[pl.ds(off, C)], ic, bufs[p][5])
            return d1, d2

        def issue_gather(p):
            ir, ic, br, bc = bufs[p][0], bufs[p][1], bufs[p][2], bufs[p][3]
            d1 = pltpu.async_copy(tr_hbm.at[ir], br, bufs[p][6])
            d2 = pltpu.async_copy(tc_hbm.at[ic], bc, bufs[p][6])
            return d1, d2

        def issue_wb(g, p):
            off = base + g * C
            return pltpu.async_copy(bufs[p][4],
                                    out_hbm.at[pl.ds(off, C), pl.ds(0, D)],
                                    bufs[p][7])

        if nsteady > 0:
            pltpu.sync_copy(row_hbm.at[pl.ds(base, C)], bufs[0][0])
            pltpu.sync_copy(col_hbm.at[pl.ds(base, C)], bufs[0][1])

            def step(gp, carry):
                g0 = 2 * gp
                g1 = g0 + 1
                dI1 = issue_idx(g1, 1)
                dG0 = issue_gather(0)
                for d in dG0:
                    d.wait()
                _row_add(bufs[0][2], bufs[0][3], bufs[0][4], C, D)
                dW0 = issue_wb(g0, 0)
                for d in dI1:
                    d.wait()
                dG1 = issue_gather(1)

                @pl.when(g0 + 2 < nfull)
                def _():
                    d1, d2 = issue_idx(g0 + 2, 0)
                    d1.wait()
                    d2.wait()
                for d in dG1:
                    d.wait()
                _row_add(bufs[1][2], bufs[1][3], bufs[1][4], C, D)
                dW1 = issue_wb(g1, 1)
                dW0.wait()
                dW1.wait()
                return carry

            lax.fori_loop(0, nsteady // 2, step, 0)

        def sync_chunk(off, n, ir, ic, br, bc, ob):
            pltpu.sync_copy(row_hbm.at[pl.ds(off, n)], ir)
            pltpu.sync_copy(col_hbm.at[pl.ds(off, n)], ic)
            d1 = pltpu.async_copy(tr_hbm.at[ir], br, bufs[0][6])
            d2 = pltpu.async_copy(tc_hbm.at[ic], bc, bufs[0][6])
            d1.wait()
            d2.wait()
            _row_add(br, bc, ob, n, D)
            pltpu.sync_copy(ob, out_hbm.at[pl.ds(off, n), pl.ds(0, D)])

        if nsteady < nfull:
            sync_chunk(base + nsteady * C, C, *bufs[0][:5])
        if rem:
            sync_chunk(base + nfull * C, rem, *tail)

    return pl.kernel(
        body,
        out_type=jax.ShapeDtypeStruct((E, 128), jnp.float32),
        mesh=plsc.VectorSubcoreMesh(**_MESH),
        scratch_types=scratch,
        compiler_params=pltpu.CompilerParams(use_tc_tiling_on_sc=False),
    )


@functools.lru_cache(maxsize=None)
def _make_scatter_add(E, Npad, D):
    Ew = E // NW
    assert Ew * NW == E
    nfull, rem = divmod(Ew, C)
    nsteady = nfull if nfull % 2 == 0 else nfull - 1
    RPS = Npad // NS
    assert RPS * NS == Npad
    scratch = []
    for _ in range(2):
        scratch += [pltpu.VMEM((C,), jnp.int32), pltpu.VMEM((C, D), jnp.float32),
                    pltpu.SemaphoreType.DMA, pltpu.SemaphoreType.DMA]
    if rem:
        scratch += [pltpu.VMEM((rem,), jnp.int32), pltpu.VMEM((rem, D), jnp.float32)]
    scratch += [pltpu.VMEM_SHARED((Npad, D), jnp.float32)]

    def body(vals_hbm, idx_hbm, zeros_hbm, out_hbm, *s):
        bufs = [s[0:4], s[4:8]]
        tail = s[8:10] if rem else None
        accum = s[-1]
        cid = lax.axis_index("c")
        sid = lax.axis_index("s")
        wid = sid * NC + cid
        base = wid * Ew

        def issue_load(g, p):
            iv, vv = bufs[p][0], bufs[p][1]
            off = base + g * C
            d1 = pltpu.async_copy(idx_hbm.at[pl.ds(off, C)], iv, bufs[p][2])
            d2 = pltpu.async_copy(vals_hbm.at[pl.ds(off, C), pl.ds(0, D)], vv,
                                  bufs[p][2])
            return d1, d2

        def issue_scat(p):
            iv, vv = bufs[p][0], bufs[p][1]
            return pltpu.async_copy(vv, accum.at[iv], bufs[p][3], add=True)

        pltpu.sync_copy(zeros_hbm, accum.at[pl.ds(sid * RPS, RPS)])
        plsc.subcore_barrier()

        if nsteady > 0:
            pltpu.sync_copy(idx_hbm.at[pl.ds(base, C)], bufs[0][0])
            pltpu.sync_copy(vals_hbm.at[pl.ds(base, C), pl.ds(0, D)], bufs[0][1])

            def step(gp, carry):
                g0 = 2 * gp
                g1 = g0 + 1
                dL1 = issue_load(g1, 1)
                dS0 = issue_scat(0)
                for d in dL1:
                    d.wait()
                dS1 = issue_scat(1)
                dS0.wait()

                @pl.when(g0 + 2 < nfull)
                def _():
                    d1, d2 = issue_load(g0 + 2, 0)
                    d1.wait()
                    d2.wait()
                dS1.wait()
                return carry

            lax.fori_loop(0, nsteady // 2, step, 0)

        def sync_chunk(off, n, iv, vv):
            pltpu.sync_copy(idx_hbm.at[pl.ds(off, n)], iv)
            pltpu.sync_copy(vals_hbm.at[pl.ds(off, n), pl.ds(0, D)], vv)
            pltpu.sync_copy(vv, accum.at[iv], add=True)

        if nsteady < nfull:
            sync_chunk(base + nsteady * C, C, bufs[0][0], bufs[0][1])
        if rem:
            sync_chunk(base + nfull * C, rem, *tail)
        plsc.subcore_barrier()
        pltpu.sync_copy(accum.at[pl.ds(sid * RPS, RPS)],
                        out_hbm.at[cid, pl.ds(sid * RPS, RPS)])

    return pl.kernel(
        body,
        out_type=jax.ShapeDtypeStruct((NC, Npad, D), jnp.float32),
        mesh=plsc.VectorSubcoreMesh(**_MESH),
        scratch_types=scratch,
        compiler_params=pltpu.CompilerParams(use_tc_tiling_on_sc=False),
    )


def _sc_gather2(t_row, t_col, row, col, D):
    E = row.shape[0]
    return _make_gather2(E, D)(t_row, t_col, row, col)


def _sc_scatter_add(vals, idx, Npad, D):
    E = idx.shape[0]
    zeros = jnp.zeros((Npad // NS, D), jnp.float32)
    return _make_scatter_add(E, Npad, D)(vals, idx, zeros)


def _pad16(n):
    return (n + 15) // 16 * 16


def _silu(x):
    return x * jax.nn.sigmoid(x)


def _elu(x):
    return jnp.where(x > 0, x, jnp.exp(jnp.minimum(x, 0.0)) - 1.0)


def _gelu(x):
    return 0.5 * x * (1.0 + lax.erf(x * (2.0 ** -0.5)))


@functools.lru_cache(maxsize=None)
def _make_edge_mlp(E, BE, with_coord):
    grid = (E // BE,)

    def body(g_ref, ea_ref, w1r_ref, w1e_ref, w2_ref, b2_ref, c1_ref, c1b_ref,
             c2_ref, out_ref):
        g = g_ref[...]
        ea = ea_ref[...]
        xd = g[:, 32:35]
        r2 = jnp.sum(xd * xd, axis=1, keepdims=True)
        t1 = (g[:, :32] + r2 * w1r_ref[...]
              + ea[:, 0:1] * w1e_ref[0:1, :] + ea[:, 1:2] * w1e_ref[1:2, :])
        m = _silu(t1)
        m = _silu(jnp.dot(m, w2_ref[...], preferred_element_type=jnp.float32)
                  + b2_ref[...])
        out_ref[:, 0:32] = m
        if with_coord:
            u = _silu(jnp.dot(m, c1_ref[...], preferred_element_type=jnp.float32)
                      + c1b_ref[...])
            s = jnp.sum(u * c2_ref[...], axis=1, keepdims=True)
            out_ref[:, 32:35] = xd * s
            out_ref[:, 35:36] = jnp.ones((BE, 1), jnp.float32)
            out_ref[:, 36:48] = jnp.zeros((BE, 12), jnp.float32)

    wspec = pl.BlockSpec(None, lambda i: (0, 0))
    return pl.pallas_call(
        body,
        grid=grid,
        in_specs=[
            pl.BlockSpec((BE, 128), lambda i: (i, 0)),
            pl.BlockSpec((BE, 2), lambda i: (i, 0)),
        ] + [wspec] * 7,
        out_specs=pl.BlockSpec((BE, 128), lambda i: (i, 0)),
        out_shape=jax.ShapeDtypeStruct((E, 128), jnp.float32),
    )


@functools.lru_cache(maxsize=None)
def _make_node_update(N, Npad, BN, mode, K=1):
    grid = (N // BN,)
    D = 48 if mode == "mid" else 32

    def body(*refs):
        if mode == "mid":
            (h_ref, x_ref, *p_refs, n1a_ref, n1b_ref, nb1_ref, n2_ref, nb2_ref,
             h_out, x_out) = refs
        else:
            (h_ref, *p_refs, n1a_ref, n1b_ref, nb1_ref, n2_ref, nb2_ref,
             wo_ref, bo_ref, out_ref) = refs
        p_refs = p_refs[:K]
        h = h_ref[...]
        s = sum(pr[1] for pr in p_refs) + p_refs[0][0]
        for pr in p_refs[1:]:
            s = s + pr[0]
        agg = s[:, :32]
        t = (jnp.dot(h, n1a_ref[...], preferred_element_type=jnp.float32)
             + jnp.dot(agg, n1b_ref[...], preferred_element_type=jnp.float32)
             + nb1_ref[...])
        h2 = h + (jnp.dot(_silu(t), n2_ref[...],
                          preferred_element_type=jnp.float32) + nb2_ref[...])
        if mode == "mid":
            cnt = jnp.maximum(s[:, 35:36], 1.0)
            h_out[...] = h2
            x_out[...] = x_ref[...] + s[:, 32:35] / cnt
        elif mode == "res":
            out_ref[...] = (jnp.dot(h2, wo_ref[...],
                                    preferred_element_type=jnp.float32)
                            + bo_ref[...])
        else:
            i = pl.program_id(0)

            @pl.when(i == 0)
            def _():
                out_ref[...] = jnp.zeros((1, 32), jnp.float32)
            out_ref[...] += jnp.sum(h2, axis=0, keepdims=True)

    wspec = pl.BlockSpec(None, lambda i: (0, 0))
    pspec = pl.BlockSpec((2, BN, D), lambda i: (0, i, 0))
    if mode == "mid":
        in_specs = [pl.BlockSpec((BN, 32), lambda i: (i, 0)),
                    pl.BlockSpec((BN, 3), lambda i: (i, 0))] + [pspec] * K + [wspec] * 5
        out_specs = [pl.BlockSpec((BN, 32), lambda i: (i, 0)),
                     pl.BlockSpec((BN, 3), lambda i: (i, 0))]
        out_shape = [jax.ShapeDtypeStruct((N, 32), jnp.float32),
                     jax.ShapeDtypeStruct((N, 3), jnp.float32)]
    else:
        in_specs = [pl.BlockSpec((BN, 32), lambda i: (i, 0))] + [pspec] * K + [wspec] * 7
        if mode == "res":
            out_specs = pl.BlockSpec((BN, 32), lambda i: (i, 0))
            out_shape = jax.ShapeDtypeStruct((N, 32), jnp.float32)
        else:
            out_specs = pl.BlockSpec((1, 32), lambda i: (0, 0))
            out_shape = jax.ShapeDtypeStruct((1, 32), jnp.float32)
    return pl.pallas_call(body, grid=grid, in_specs=in_specs,
                          out_specs=out_specs, out_shape=out_shape)


@functools.lru_cache(maxsize=None)
def _make_tables(N, BN):
    grid = (N // BN,)

    def body(h_ref, x_ref, wa_ref, ba_ref, wb_ref, tr_ref, tc_ref):
        h = h_ref[...]
        x = x_ref[...]
        z = jnp.zeros((BN, 13), jnp.float32)
        tr_ref[:, 0:32] = (jnp.dot(h, wa_ref[...],
                                   preferred_element_type=jnp.float32)
                           + ba_ref[...])
        tr_ref[:, 32:35] = x
        tr_ref[:, 35:48] = z
        tc_ref[:, 0:32] = jnp.dot(h, wb_ref[...],
                                  preferred_element_type=jnp.float32)
        tc_ref[:, 32:35] = -x
        tc_ref[:, 35:48] = z

    wspec = pl.BlockSpec(None, lambda i: (0, 0))
    nspec = pl.BlockSpec((BN, 48), lambda i: (i, 0))
    return pl.pallas_call(
        body,
        grid=grid,
        in_specs=[pl.BlockSpec((BN, 32), lambda i: (i, 0)),
                  pl.BlockSpec((BN, 3), lambda i: (i, 0))] + [wspec] * 3,
        out_specs=[nspec, nspec],
        out_shape=[jax.ShapeDtypeStruct((N, 48), jnp.float32),
                   jax.ShapeDtypeStruct((N, 48), jnp.float32)],
    )


@functools.lru_cache(maxsize=None)
def _make_r2a_mm(M, K, BM):
    grid = (M // BM,)

    def body(a_ref, b_ref, out_ref):
        out_ref[...] = jnp.dot(a_ref[...], b_ref[...],
                               preferred_element_type=jnp.float32)

    return pl.pallas_call(
        body,
        grid=grid,
        in_specs=[pl.BlockSpec((BM, K), lambda i: (i, 0)),
                  pl.BlockSpec(None, lambda i: (0, 0))],
        out_specs=pl.BlockSpec((BM, 32), lambda i: (i, 0)),
        out_shape=jax.ShapeDtypeStruct((M, 32), jnp.float32),
    )


@functools.lru_cache(maxsize=None)
def _make_proj(N, BN):
    grid = (N // BN,)

    def body(n0_ref, m2_ref, w00_ref, b00_ref, w00b_ref, b00b_ref,
             w01_ref, b01_ref, w01b_ref, b01b_ref, w1_ref, b1_ref,
             wb_ref, bemb_ref, out_ref):
        n0 = n0_ref[...]
        a = _elu(jnp.dot(n0[:, 1:22], w00_ref[...],
                         preferred_element_type=jnp.float32) + b00_ref[...])
        h00 = jnp.dot(a, w00b_ref[...],
                      preferred_element_type=jnp.float32) + b00b_ref[...]
        b = _elu(jnp.dot(n0[:, 22:87], w01_ref[...],
                         preferred_element_type=jnp.float32) + b01_ref[...])
        h01 = jnp.dot(b, w01b_ref[...],
                      preferred_element_type=jnp.float32) + b01b_ref[...]
        w1 = w1_ref[...]
        t = (n0[:, 0:1] * w1[0:1, :]
             + jnp.dot(h00, w1[1:16, :], preferred_element_type=jnp.float32)
             + jnp.dot(h01, w1[16:31, :], preferred_element_type=jnp.float32)
             + jnp.dot(n0[:, 87:151], w1[31:95, :],
                       preferred_element_type=jnp.float32)
             + b1_ref[...])
        ha = _elu(t)
        out_ref[...] = (m2_ref[...]
                        + jnp.dot(ha, wb_ref[...],
                                  preferred_element_type=jnp.float32)
                        + bemb_ref[...])

    wspec = pl.BlockSpec(None, lambda i: (0, 0))
    return pl.pallas_call(
        body,
        grid=grid,
        in_specs=[pl.BlockSpec((BN, 151), lambda i: (i, 0)),
                  pl.BlockSpec((BN, 32), lambda i: (i, 0))] + [wspec] * 12,
        out_specs=pl.BlockSpec((BN, 32), lambda i: (i, 0)),
        out_shape=jax.ShapeDtypeStruct((N, 32), jnp.float32),
    )


@functools.lru_cache(maxsize=None)
def _make_res_pre(N):
    def body(f_ref, w1_ref, b1_ref, wf_ref, bf_ref, out_ref):
        a = _elu(jnp.dot(f_ref[...], w1_ref[...],
                         preferred_element_type=jnp.float32) + b1_ref[...])
        out_ref[...] = jnp.dot(a, wf_ref[...],
                               preferred_element_type=jnp.float32) + bf_ref[...]

    return pl.pallas_call(
        body,
        out_shape=jax.ShapeDtypeStruct((N, 32), jnp.float32),
    )


@functools.lru_cache(maxsize=None)
def _make_head(n_nodes):
    def body(hs_ref, we_ref, be_ref, w1_ref, b1_ref, w2_ref, b2_ref,
             w3_ref, b3_ref, out_ref):
        pooled = (jnp.dot(hs_ref[...] * (1.0 / n_nodes), we_ref[...],
                          preferred_element_type=jnp.float32) + be_ref[...])
        z = _gelu(jnp.dot(pooled, w1_ref[...],
                          preferred_element_type=jnp.float32) + b1_ref[...])
        z = _gelu(jnp.dot(z, w2_ref[...],
                          preferred_element_type=jnp.float32) + b2_ref[...])
        out_ref[...] = jnp.dot(z, w3_ref[...],
                               preferred_element_type=jnp.float32) + b3_ref[...]

    return pl.pallas_call(
        body,
        out_shape=jax.ShapeDtypeStruct((1, 9), jnp.float32),
    )


def _row(b):
    return b.reshape(1, -1)


def _egnn_sc(p, h, x, row, col, edge_attr, n, BN, BE, final_mode, wo, bo, K=1):
    E = row.shape[0]
    Es = E // K
    npad = _pad16(n)
    lp1, lp2 = p["layers"]

    def layer(lp, h, x, with_coord, D):
        W1 = lp["e1"]["w"]
        tr, tc = _make_tables(n, BN)(h, x, W1[:32], _row(lp["e1"]["b"]),
                                     W1[32:64])
        parts = []
        for k in range(K):
            sl = slice(k * Es, (k + 1) * Es)
            g = _sc_gather2(tr, tc, row[sl], col[sl], 48)
            v = _make_edge_mlp(Es, BE, with_coord)(
                g, edge_attr[sl], W1[64:65], W1[65:67],
                lp["e2"]["w"], _row(lp["e2"]["b"]),
                lp["c1"]["w"], _row(lp["c1"]["b"]), lp["c2w"].reshape(1, 32))
            parts.append(_sc_scatter_add(v, row[sl], npad, D))
        return parts

    nw = lambda lp: (lp["n1"]["w"][:32], lp["n1"]["w"][32:],
                     _row(lp["n1"]["b"]), lp["n2"]["w"], _row(lp["n2"]["b"]))

    parts = layer(lp1, h, x, True, 48)
    h, x = _make_node_update(n, npad, BN, "mid", K)(h, x, *parts, *nw(lp1))

    parts = layer(lp2, h, x, False, 32)
    return _make_node_update(n, npad, BN, final_mode, K)(h, *parts, *nw(lp2),
                                                         wo, bo)


def kernel(atm_node_feat, atm_coords, atm_edge_index, atm_edge_attr, res_node_feat, res_coords, res_edge_index, res_edge_attr, r2a, params):
    p = params
    res_egnn, atm_egnn = p["res_egnn"], p["atm_egnn"]
    wf = p["res_lin2"]["w"] @ res_egnn["emb_in"]["w"]
    bf = p["res_lin2"]["b"] @ res_egnn["emb_in"]["w"] + res_egnn["emb_in"]["b"]
    wet = res_egnn["emb_out"]["w"] @ atm_egnn["emb_in"]["w"][:32]
    bet = res_egnn["emb_out"]["b"] @ atm_egnn["emb_in"]["w"][:32]

    h0_res = _make_res_pre(N_RES)(res_node_feat, p["res_lin1"]["w"],
                                  _row(p["res_lin1"]["b"]), wf, _row(bf))
    emb2 = _egnn_sc(res_egnn, h0_res, res_coords,
                    res_edge_index[0], res_edge_index[1], res_edge_attr,
                    N_RES, 1000, 8000, "res", wet, _row(bet))
    m2 = _make_r2a_mm(N_ATM, N_RES, 1000)(r2a, emb2)
    h0_atm = _make_proj(N_ATM, 2000)(
        atm_node_feat, m2,
        p["lin00"]["w"], _row(p["lin00"]["b"]),
        p["lin00b"]["w"], _row(p["lin00b"]["b"]),
        p["lin01"]["w"], _row(p["lin01"]["b"]),
        p["lin01b"]["w"], _row(p["lin01b"]["b"]),
        p["lin1_atm"]["w"], _row(p["lin1_atm"]["b"]),
        atm_egnn["emb_in"]["w"][32:], _row(atm_egnn["emb_in"]["b"]))
    hsum = _egnn_sc(atm_egnn, h0_atm, atm_coords,
                    atm_edge_index[0], atm_edge_index[1], atm_edge_attr,
                    N_ATM, 2000, 8000, "atom",
                    atm_egnn["emb_out"]["w"], _row(atm_egnn["emb_out"]["b"]),
                    K=10)
    return _make_head(N_ATM)(
        hsum, atm_egnn["emb_out"]["w"], _row(atm_egnn["emb_out"]["b"]),
        p["cls1"]["w"], _row(p["cls1"]["b"]),
        p["cls2"]["w"], _row(p["cls2"]["b"]),
        p["cls3"]["w"], _row(p["cls3"]["b"]))


# speedup vs baseline: 1.0960x; 1.0960x over previous
import functools

import jax
import jax.numpy as jnp
from jax import lax
from jax.experimental import pallas as pl
from jax.experimental.pallas import tpu as pltpu
from jax.experimental.pallas import tpu_sc as plsc

N_ATM = 10000
N_RES = 1000

NC = 2
NS = 16
NW = NC * NS
C = 128

_MESH = dict(core_axis_name="c", subcore_axis_name="s")


def _row_add(bufr, bufc, outb, nrows, D):
    def rbody(i, carry):
        for j in range(D // 16):
            sl = pl.ds(j * 16, 16)
            outb[i, sl] = bufr[i, sl] + bufc[i, sl]
        return carry
    lax.fori_loop(0, nrows, rbody, 0)


@functools.lru_cache(maxsize=None)
def _make_gather2(E, D):
    Ew = E // NW
    assert Ew * NW == E
    nfull, rem = divmod(Ew, C)
    nsteady = nfull if nfull % 2 == 0 else nfull - 1
    scratch = []
    for _ in range(2):
        scratch += [pltpu.VMEM((C,), jnp.int32), pltpu.VMEM((C,), jnp.int32),
                    pltpu.VMEM((C, D), jnp.float32), pltpu.VMEM((C, D), jnp.float32),
                    pltpu.VMEM((C, D), jnp.float32),
                    pltpu.SemaphoreType.DMA, pltpu.SemaphoreType.DMA,
                    pltpu.SemaphoreType.DMA]
    if rem:
        scratch += [pltpu.VMEM((rem,), jnp.int32), pltpu.VMEM((rem,), jnp.int32),
                    pltpu.VMEM((rem, D), jnp.float32), pltpu.VMEM((rem, D), jnp.float32),
                    pltpu.VMEM((rem, D), jnp.float32)]

    def body(tr_hbm, tc_hbm, row_hbm, col_hbm, out_hbm, *s):
        bufs = [s[0:8], s[8:16]]
        tail = s[16:] if rem else None
        wid = lax.axis_index("s") * NC + lax.axis_index("c")
        base = wid * Ew

        def issue_idx(g, p):
            ir, ic = bufs[p][0], bufs[p][1]
            off = base + g * C
            d1 = pltpu.async_copy(row_hbm.at[pl.ds(off, C)], ir, bufs[p][5])
            d2 = pltpu.async_copy(col_hbm.at[pl.ds(off, C)], ic, bufs[p][5])
            return d1, d2

        def issue_gather(p):
            ir, ic, br, bc = bufs[p][0], bufs[p][1], bufs[p][2], bufs[p][3]
            d1 = pltpu.async_copy(tr_hbm.at[ir], br, bufs[p][6])
            d2 = pltpu.async_copy(tc_hbm.at[ic], bc, bufs[p][6])
            return d1, d2

        def issue_wb(g, p):
            off = base + g * C
            return pltpu.async_copy(bufs[p][4],
                                    out_hbm.at[pl.ds(off, C), pl.ds(0, D)],
                                    bufs[p][7])

        if nsteady > 0:
            pltpu.sync_copy(row_hbm.at[pl.ds(base, C)], bufs[0][0])
            pltpu.sync_copy(col_hbm.at[pl.ds(base, C)], bufs[0][1])

            def step(gp, carry):
                g0 = 2 * gp
                g1 = g0 + 1
                dI1 = issue_idx(g1, 1)
                dG0 = issue_gather(0)
                for d in dG0:
                    d.wait()
                _row_add(bufs[0][2], bufs[0][3], bufs[0][4], C, D)
                dW0 = issue_wb(g0, 0)
                for d in dI1:
                    d.wait()
                dG1 = issue_gather(1)

                @pl.when(g0 + 2 < nfull)
                def _():
                    d1, d2 = issue_idx(g0 + 2, 0)
                    d1.wait()
                    d2.wait()
                for d in dG1:
                    d.wait()
                _row_add(bufs[1][2], bufs[1][3], bufs[1][4], C, D)
                dW1 = issue_wb(g1, 1)
                dW0.wait()
                dW1.wait()
                return carry

            lax.fori_loop(0, nsteady // 2, step, 0)

        def sync_chunk(off, n, ir, ic, br, bc, ob):
            pltpu.sync_copy(row_hbm.at[pl.ds(off, n)], ir)
            pltpu.sync_copy(col_hbm.at[pl.ds(off, n)], ic)
            d1 = pltpu.async_copy(tr_hbm.at[ir], br, bufs[0][6])
            d2 = pltpu.async_copy(tc_hbm.at[ic], bc, bufs[0][6])
            d1.wait()
            d2.wait()
            _row_add(br, bc, ob, n, D)
            pltpu.sync_copy(ob, out_hbm.at[pl.ds(off, n), pl.ds(0, D)])

        if nsteady < nfull:
            sync_chunk(base + nsteady * C, C, *bufs[0][:5])
        if rem:
            sync_chunk(base + nfull * C, rem, *tail)

    return pl.kernel(
        body,
        out_type=jax.ShapeDtypeStruct((E, 128), jnp.float32),
        mesh=plsc.VectorSubcoreMesh(**_MESH),
        scratch_types=scratch,
        compiler_params=pltpu.CompilerParams(use_tc_tiling_on_sc=False),
    )


@functools.lru_cache(maxsize=None)
def _make_scatter_add(E, Npad, D):
    Ew = E // NW
    assert Ew * NW == E
    nfull, rem = divmod(Ew, C)
    nsteady = nfull if nfull % 2 == 0 else nfull - 1
    RPS = Npad // NS
    assert RPS * NS == Npad
    scratch = []
    for _ in range(2):
        scratch += [pltpu.VMEM((C,), jnp.int32), pltpu.VMEM((C, D), jnp.float32),
                    pltpu.SemaphoreType.DMA, pltpu.SemaphoreType.DMA]
    if rem:
        scratch += [pltpu.VMEM((rem,), jnp.int32), pltpu.VMEM((rem, D), jnp.float32)]
    scratch += [pltpu.VMEM_SHARED((Npad, D), jnp.float32)]

    def body(vals_hbm, idx_hbm, zeros_hbm, out_hbm, *s):
        bufs = [s[0:4], s[4:8]]
        tail = s[8:10] if rem else None
        accum = s[-1]
        cid = lax.axis_index("c")
        sid = lax.axis_index("s")
        wid = sid * NC + cid
        base = wid * Ew

        def issue_load(g, p):
            iv, vv = bufs[p][0], bufs[p][1]
            off = base + g * C
            d1 = pltpu.async_copy(idx_hbm.at[pl.ds(off, C)], iv, bufs[p][2])
            d2 = pltpu.async_copy(vals_hbm.at[pl.ds(off, C), pl.ds(0, D)], vv,
                                  bufs[p][2])
            return d1, d2

        def issue_scat(p):
            iv, vv = bufs[p][0], bufs[p][1]
            return pltpu.async_copy(vv, accum.at[iv], bufs[p][3], add=True)

        pltpu.sync_copy(zeros_hbm, accum.at[pl.ds(sid * RPS, RPS)])
        plsc.subcore_barrier()

        if nsteady > 0:
            pltpu.sync_copy(idx_hbm.at[pl.ds(base, C)], bufs[0][0])
            pltpu.sync_copy(vals_hbm.at[pl.ds(base, C), pl.ds(0, D)], bufs[0][1])

            def step(gp, carry):
                g0 = 2 * gp
                g1 = g0 + 1
                dL1 = issue_load(g1, 1)
                dS0 = issue_scat(0)
                for d in dL1:
                    d.wait()
                dS1 = issue_scat(1)
                dS0.wait()

                @pl.when(g0 + 2 < nfull)
                def _():
                    d1, d2 = issue_load(g0 + 2, 0)
                    d1.wait()
                    d2.wait()
                dS1.wait()
                return carry

            lax.fori_loop(0, nsteady // 2, step, 0)

        def sync_chunk(off, n, iv, vv):
            pltpu.sync_copy(idx_hbm.at[pl.ds(off, n)], iv)
            pltpu.sync_copy(vals_hbm.at[pl.ds(off, n), pl.ds(0, D)], vv)
            pltpu.sync_copy(vv, accum.at[iv], add=True)

        if nsteady < nfull:
            sync_chunk(base + nsteady * C, C, bufs[0][0], bufs[0][1])
        if rem:
            sync_chunk(base + nfull * C, rem, *tail)
        plsc.subcore_barrier()
        pltpu.sync_copy(accum.at[pl.ds(sid * RPS, RPS)],
                        out_hbm.at[cid, pl.ds(sid * RPS, RPS)])

    return pl.kernel(
        body,
        out_type=jax.ShapeDtypeStruct((NC, Npad, D), jnp.float32),
        mesh=plsc.VectorSubcoreMesh(**_MESH),
        scratch_types=scratch,
        compiler_params=pltpu.CompilerParams(use_tc_tiling_on_sc=False),
    )


def _sc_gather2(t_row, t_col, row, col, D):
    E = row.shape[0]
    return _make_gather2(E, D)(t_row, t_col, row, col)


def _sc_scatter_add(vals, idx, Npad, D):
    E = idx.shape[0]
    zeros = jnp.zeros((Npad // NS, D), jnp.float32)
    return _make_scatter_add(E, Npad, D)(vals, idx, zeros)


def _pad16(n):
    return (n + 15) // 16 * 16


def _silu(x):
    return x * jax.nn.sigmoid(x)


def _elu(x):
    return jnp.where(x > 0, x, jnp.exp(jnp.minimum(x, 0.0)) - 1.0)


def _gelu(x):
    return 0.5 * x * (1.0 + lax.erf(x * (2.0 ** -0.5)))


@functools.lru_cache(maxsize=None)
def _make_edge_mlp(E, BE, with_coord):
    grid = (E // BE,)

    def body(g_ref, ea_ref, w1r_ref, w1e_ref, w2_ref, b2_ref, c1_ref, c1b_ref,
             c2_ref, out_ref):
        g = g_ref[...]
        ea = ea_ref[...]
        xd = g[:, 32:35]
        r2 = jnp.sum(xd * xd, axis=1, keepdims=True)
        t1 = (g[:, :32] + r2 * w1r_ref[...]
              + ea[:, 0:1] * w1e_ref[0:1, :] + ea[:, 1:2] * w1e_ref[1:2, :])
        m = _silu(t1)
        m = _silu(jnp.dot(m, w2_ref[...], preferred_element_type=jnp.float32)
                  + b2_ref[...])
        out_ref[:, 0:32] = m
        if with_coord:
            u = _silu(jnp.dot(m, c1_ref[...], preferred_element_type=jnp.float32)
                      + c1b_ref[...])
            s = jnp.sum(u * c2_ref[...], axis=1, keepdims=True)
            out_ref[:, 32:35] = xd * s
            out_ref[:, 35:36] = jnp.ones((BE, 1), jnp.float32)
            out_ref[:, 36:48] = jnp.zeros((BE, 12), jnp.float32)

    wspec = pl.BlockSpec(None, lambda i: (0, 0))
    return pl.pallas_call(
        body,
        grid=grid,
        in_specs=[
            pl.BlockSpec((BE, 128), lambda i: (i, 0)),
            pl.BlockSpec((BE, 2), lambda i: (i, 0)),
        ] + [wspec] * 7,
        out_specs=pl.BlockSpec((BE, 128), lambda i: (i, 0)),
        out_shape=jax.ShapeDtypeStruct((E, 128), jnp.float32),
    )


@functools.lru_cache(maxsize=None)
def _make_node_update(N, Npad, BN, mode, K=1):
    grid = (N // BN,)
    D = 48 if mode == "mid" else 32

    def body(*refs):
        if mode == "mid":
            (h_ref, x_ref, *p_refs, n1a_ref, n1b_ref, nb1_ref, n2_ref, nb2_ref,
             h_out, x_out) = refs
        else:
            (h_ref, *p_refs, n1a_ref, n1b_ref, nb1_ref, n2_ref, nb2_ref,
             wo_ref, bo_ref, out_ref) = refs
        p_refs = p_refs[:K]
        h = h_ref[...]
        s = sum(pr[1] for pr in p_refs) + p_refs[0][0]
        for pr in p_refs[1:]:
            s = s + pr[0]
        agg = s[:, :32]
        t = (jnp.dot(h, n1a_ref[...], preferred_element_type=jnp.float32)
             + jnp.dot(agg, n1b_ref[...], preferred_element_type=jnp.float32)
             + nb1_ref[...])
        h2 = h + (jnp.dot(_silu(t), n2_ref[...],
                          preferred_element_type=jnp.float32) + nb2_ref[...])
        if mode == "mid":
            cnt = jnp.maximum(s[:, 35:36], 1.0)
            h_out[...] = h2
            x_out[...] = x_ref[...] + s[:, 32:35] / cnt
        elif mode == "res":
            out_ref[...] = (jnp.dot(h2, wo_ref[...],
                                    preferred_element_type=jnp.float32)
                            + bo_ref[...])
        else:
            i = pl.program_id(0)

            @pl.when(i == 0)
            def _():
                out_ref[...] = jnp.zeros((1, 32), jnp.float32)
            out_ref[...] += jnp.sum(h2, axis=0, keepdims=True)

    wspec = pl.BlockSpec(None, lambda i: (0, 0))
    pspec = pl.BlockSpec((2, BN, D), lambda i: (0, i, 0))
    if mode == "mid":
        in_specs = [pl.BlockSpec((BN, 32), lambda i: (i, 0)),
                    pl.BlockSpec((BN, 3), lambda i: (i, 0))] + [pspec] * K + [wspec] * 5
        out_specs = [pl.BlockSpec((BN, 32), lambda i: (i, 0)),
                     pl.BlockSpec((BN, 3), lambda i: (i, 0))]
        out_shape = [jax.ShapeDtypeStruct((N, 32), jnp.float32),
                     jax.ShapeDtypeStruct((N, 3), jnp.float32)]
    else:
        in_specs = [pl.BlockSpec((BN, 32), lambda i: (i, 0))] + [pspec] * K + [wspec] * 7
        if mode == "res":
            out_specs = pl.BlockSpec((BN, 32), lambda i: (i, 0))
            out_shape = jax.ShapeDtypeStruct((N, 32), jnp.float32)
        else:
            out_specs = pl.BlockSpec((1, 32), lambda i: (0, 0))
            out_shape = jax.ShapeDtypeStruct((1, 32), jnp.float32)
    return pl.pallas_call(body, grid=grid, in_specs=in_specs,
                          out_specs=out_specs, out_shape=out_shape)


@functools.lru_cache(maxsize=None)
def _make_tables(N, BN):
    grid = (N // BN,)

    def body(h_ref, x_ref, wa_ref, ba_ref, wb_ref, tr_ref, tc_ref):
        h = h_ref[...]
        x = x_ref[...]
        z = jnp.zeros((BN, 13), jnp.float32)
        tr_ref[:, 0:32] = (jnp.dot(h, wa_ref[...],
                                   preferred_element_type=jnp.float32)
                           + ba_ref[...])
        tr_ref[:, 32:35] = x
        tr_ref[:, 35:48] = z
        tc_ref[:, 0:32] = jnp.dot(h, wb_ref[...],
                                  preferred_element_type=jnp.float32)
        tc_ref[:, 32:35] = -x
        tc_ref[:, 35:48] = z

    wspec = pl.BlockSpec(None, lambda i: (0, 0))
    nspec = pl.BlockSpec((BN, 48), lambda i: (i, 0))
    return pl.pallas_call(
        body,
        grid=grid,
        in_specs=[pl.BlockSpec((BN, 32), lambda i: (i, 0)),
                  pl.BlockSpec((BN, 3), lambda i: (i, 0))] + [wspec] * 3,
        out_specs=[nspec, nspec],
        out_shape=[jax.ShapeDtypeStruct((N, 48), jnp.float32),
                   jax.ShapeDtypeStruct((N, 48), jnp.float32)],
    )


@functools.lru_cache(maxsize=None)
def _make_r2a_mm(M, K, BM):
    grid = (M // BM,)

    def body(a_ref, b_ref, out_ref):
        out_ref[...] = jnp.dot(a_ref[...], b_ref[...],
                               preferred_element_type=jnp.float32)

    return pl.pallas_call(
        body,
        grid=grid,
        in_specs=[pl.BlockSpec((BM, K), lambda i: (i, 0)),
                  pl.BlockSpec(None, lambda i: (0, 0))],
        out_specs=pl.BlockSpec((BM, 32), lambda i: (i, 0)),
        out_shape=jax.ShapeDtypeStruct((M, 32), jnp.float32),
    )


@functools.lru_cache(maxsize=None)
def _make_proj(N, BN):
    grid = (N // BN,)

    def body(n0_ref, m2_ref, w00_ref, b00_ref, w00b_ref, b00b_ref,
             w01_ref, b01_ref, w01b_ref, b01b_ref, w1_ref, b1_ref,
             wb_ref, bemb_ref, out_ref):
        n0 = n0_ref[...]
        a = _elu(jnp.dot(n0[:, 1:22], w00_ref[...],
                         preferred_element_type=jnp.float32) + b00_ref[...])
        h00 = jnp.dot(a, w00b_ref[...],
                      preferred_element_type=jnp.float32) + b00b_ref[...]
        b = _elu(jnp.dot(n0[:, 22:87], w01_ref[...],
                         preferred_element_type=jnp.float32) + b01_ref[...])
        h01 = jnp.dot(b, w01b_ref[...],
                      preferred_element_type=jnp.float32) + b01b_ref[...]
        w1 = w1_ref[...]
        t = (n0[:, 0:1] * w1[0:1, :]
             + jnp.dot(h00, w1[1:16, :], preferred_element_type=jnp.float32)
             + jnp.dot(h01, w1[16:31, :], preferred_element_type=jnp.float32)
             + jnp.dot(n0[:, 87:151], w1[31:95, :],
                       preferred_element_type=jnp.float32)
             + b1_ref[...])
        ha = _elu(t)
        out_ref[...] = (m2_ref[...]
                        + jnp.dot(ha, wb_ref[...],
                                  preferred_element_type=jnp.float32)
                        + bemb_ref[...])

    wspec = pl.BlockSpec(None, lambda i: (0, 0))
    return pl.pallas_call(
        body,
        grid=grid,
        in_specs=[pl.BlockSpec((BN, 151), lambda i: (i, 0)),
                  pl.BlockSpec((BN, 32), lambda i: (i, 0))] + [wspec] * 12,
        out_specs=pl.BlockSpec((BN, 32), lambda i: (i, 0)),
        out_shape=jax.ShapeDtypeStruct((N, 32), jnp.float32),
    )


@functools.lru_cache(maxsize=None)
def _make_res_pre(N):
    def body(f_ref, w1_ref, b1_ref, wf_ref, bf_ref, out_ref):
        a = _elu(jnp.dot(f_ref[...], w1_ref[...],
                         preferred_element_type=jnp.float32) + b1_ref[...])
        out_ref[...] = jnp.dot(a, wf_ref[...],
                               preferred_element_type=jnp.float32) + bf_ref[...]

    return pl.pallas_call(
        body,
        out_shape=jax.ShapeDtypeStruct((N, 32), jnp.float32),
    )


@functools.lru_cache(maxsize=None)
def _make_head(n_nodes):
    def body(hs_ref, we_ref, be_ref, w1_ref, b1_ref, w2_ref, b2_ref,
             w3_ref, b3_ref, out_ref):
        pooled = (jnp.dot(hs_ref[...] * (1.0 / n_nodes), we_ref[...],
                          preferred_element_type=jnp.float32) + be_ref[...])
        z = _gelu(jnp.dot(pooled, w1_ref[...],
                          preferred_element_type=jnp.float32) + b1_ref[...])
        z = _gelu(jnp.dot(z, w2_ref[...],
                          preferred_element_type=jnp.float32) + b2_ref[...])
        out_ref[...] = jnp.dot(z, w3_ref[...],
                               preferred_element_type=jnp.float32) + b3_ref[...]

    return pl.pallas_call(
        body,
        out_shape=jax.ShapeDtypeStruct((1, 9), jnp.float32),
    )


def _row(b):
    return b.reshape(1, -1)


def _egnn_sc(p, h, x, row, col, edge_attr, n, BN, BE, final_mode, wo, bo, K=1):
    E = row.shape[0]
    Es = E // K
    npad = _pad16(n)
    lp1, lp2 = p["layers"]

    def layer(lp, h, x, with_coord, D):
        W1 = lp["e1"]["w"]
        tr, tc = _make_tables(n, BN)(h, x, W1[:32], _row(lp["e1"]["b"]),
                                     W1[32:64])
        parts = []
        for k in range(K):
            sl = slice(k * Es, (k + 1) * Es)
            g = _sc_gather2(tr, tc, row[sl], col[sl], 48)
            v = _make_edge_mlp(Es, BE, with_coord)(
                g, edge_attr[sl], W1[64:65], W1[65:67],
                lp["e2"]["w"], _row(lp["e2"]["b"]),
                lp["c1"]["w"], _row(lp["c1"]["b"]), lp["c2w"].reshape(1, 32))
            parts.append(_sc_scatter_add(v, row[sl], npad, D))
        return parts

    nw = lambda lp: (lp["n1"]["w"][:32], lp["n1"]["w"][32:],
                     _row(lp["n1"]["b"]), lp["n2"]["w"], _row(lp["n2"]["b"]))

    parts = layer(lp1, h, x, True, 48)
    h, x = _make_node_update(n, npad, BN, "mid", K)(h, x, *parts, *nw(lp1))

    parts = layer(lp2, h, x, False, 32)
    return _make_node_update(n, npad, BN, final_mode, K)(h, *parts, *nw(lp2),
                                                         wo, bo)


def kernel(atm_node_feat, atm_coords, atm_edge_index, atm_edge_attr, res_node_feat, res_coords, res_edge_index, res_edge_attr, r2a, params):
    p = params
    res_egnn, atm_egnn = p["res_egnn"], p["atm_egnn"]
    wf = p["res_lin2"]["w"] @ res_egnn["emb_in"]["w"]
    bf = p["res_lin2"]["b"] @ res_egnn["emb_in"]["w"] + res_egnn["emb_in"]["b"]
    wet = res_egnn["emb_out"]["w"] @ atm_egnn["emb_in"]["w"][:32]
    bet = res_egnn["emb_out"]["b"] @ atm_egnn["emb_in"]["w"][:32]

    h0_res = _make_res_pre(N_RES)(res_node_feat, p["res_lin1"]["w"],
                                  _row(p["res_lin1"]["b"]), wf, _row(bf))
    emb2 = _egnn_sc(res_egnn, h0_res, res_coords,
                    res_edge_index[0], res_edge_index[1], res_edge_attr,
                    N_RES, 1000, 8000, "res", wet, _row(bet))
    m2 = _make_r2a_mm(N_ATM, N_RES, 1000)(r2a, emb2)
    h0_atm = _make_proj(N_ATM, 2000)(
        atm_node_feat, m2,
        p["lin00"]["w"], _row(p["lin00"]["b"]),
        p["lin00b"]["w"], _row(p["lin00b"]["b"]),
        p["lin01"]["w"], _row(p["lin01"]["b"]),
        p["lin01b"]["w"], _row(p["lin01b"]["b"]),
        p["lin1_atm"]["w"], _row(p["lin1_atm"]["b"]),
        atm_egnn["emb_in"]["w"][32:], _row(atm_egnn["emb_in"]["b"]))
    hsum = _egnn_sc(atm_egnn, h0_atm, atm_coords,
                    atm_edge_index[0], atm_edge_index[1], atm_edge_attr,
                    N_ATM, 2000, 8000, "atom",
                    atm_egnn["emb_out"]["w"], _row(atm_egnn["emb_out"]["b"]),
                    K=5)
    return _make_head(N_ATM)(
        hsum, atm_egnn["emb_out"]["w"], _row(atm_egnn["emb_out"]["b"]),
        p["cls1"]["w"], _row(p["cls1"]["b"]),
        p["cls2"]["w"], _row(p["cls2"]["b"]),
        p["cls3"]["w"], _row(p["cls3"]["b"]))
